# baseline (speedup 1.0000x reference)
import functools

import jax
import jax.numpy as jnp
from jax import lax
from jax.experimental import pallas as pl
from jax.experimental.pallas import tpu as pltpu
from jax.experimental.pallas import tpu_sc as plsc

_NC = 2
_NS = 16
_NW = _NC * _NS
_CHUNK = 128
_TBLK = 128
_NBUF = 2


def _transpose_block(in_ref, out_ref, nrows, dim, iota16):
    for r16 in range(nrows // 16):
        for j in range(dim):
            src = plsc.load_gather(
                in_ref, [jnp.full((16,), j, jnp.int32), r16 * 16 + iota16]
            )
            flat = r16 * 16 * dim + j + iota16 * dim
            plsc.store_scatter(
                out_ref,
                [lax.shift_right_logical(flat, jnp.int32(7)),
                 lax.bitwise_and(flat, jnp.int32(127))],
                src,
            )


@functools.lru_cache(maxsize=None)
def _build_transpose(vocab, dim):
    n_full = vocab // _TBLK
    k_max = -(-n_full // _NW)
    srows = _TBLK * dim // 128
    mesh = plsc.VectorSubcoreMesh(core_axis_name="c", subcore_axis_name="s")

    @functools.partial(
        pl.kernel,
        mesh=mesh,
        compiler_params=pltpu.CompilerParams(
            use_tc_tiling_on_sc=True, needs_layout_passes=False
        ),
        out_type=jax.ShapeDtypeStruct((n_full * srows, 128), jnp.float32),
        scratch_types=[
            pltpu.VMEM((_NBUF, dim, _TBLK), jnp.float32),
            pltpu.VMEM((_NBUF, srows, 128), jnp.float32),
            pltpu.SemaphoreType.DMA,
            pltpu.SemaphoreType.DMA,
            pltpu.SemaphoreType.DMA,
            pltpu.SemaphoreType.DMA,
        ],
    )
    def _transpose(tt_hbm, s_hbm, in_v, out_v, gi0, gi1, go0, go1):
        wid = lax.axis_index("s") * _NC + lax.axis_index("c")
        gis = (gi0, gi1)
        gos = (go0, go1)
        iota16 = lax.iota(jnp.int32, 16)

        def start_in(b, slot):
            pltpu.async_copy(
                tt_hbm.at[:, pl.ds(b * _TBLK, _TBLK)], in_v.at[slot], gis[slot]
            )

        def wait_out(slot):
            pltpu.make_async_copy(
                out_v.at[slot], s_hbm.at[pl.ds(0, srows)], gos[slot]
            ).wait()

        def body(b, slot):
            pltpu.make_async_copy(
                tt_hbm.at[:, pl.ds(b * _TBLK, _TBLK)], in_v.at[slot], gis[slot]
            ).wait()

            @pl.when(b >= _NW * _NBUF)
            def _():
                wait_out(slot)

            _transpose_block(in_v.at[slot], out_v.at[slot], _TBLK, dim, iota16)
            pltpu.async_copy(
                out_v.at[slot], s_hbm.at[pl.ds(b * srows, srows)], gos[slot]
            )

        @pl.when(wid < n_full)
        def _():
            start_in(wid, 0)

        @pl.loop(0, k_max, step=_NBUF)
        def _blocks(k):
            for b_ in range(_NBUF):
                b = wid + (k + b_) * _NW
                nxt = b + _NW
                slot = b_

                @pl.when(nxt < n_full)
                def _(nxt=nxt, slot=slot):
                    start_in(nxt, (slot + 1) % _NBUF)

                @pl.when(b < n_full)
                def _(b=b, slot=slot):
                    body(b, slot)

        for slot in range(_NBUF):
            @pl.when(wid + slot * _NW < n_full)
            def _(slot=slot):
                wait_out(slot)

    return _transpose


@functools.lru_cache(maxsize=None)
def _build_gather(b_pad, vocab, dim):
    n_chunks = b_pad // _CHUNK
    k_max = n_chunks // _NW
    rpw = 128 // dim
    vocab_cut = (vocab // _TBLK) * _TBLK
    smax = vocab_cut * dim // 128 - 1
    ntail = vocab - vocab_cut
    trows = ntail * dim // 128
    mesh = plsc.VectorSubcoreMesh(core_axis_name="c", subcore_axis_name="s")

    @functools.partial(
        pl.kernel,
        mesh=mesh,
        compiler_params=pltpu.CompilerParams(
            use_tc_tiling_on_sc=True, needs_layout_passes=False
        ),
        out_type=jax.ShapeDtypeStruct((dim, b_pad), jnp.float32),
        scratch_types=[
            pltpu.VMEM((32, _CHUNK), jnp.int32),
            pltpu.VMEM((_NBUF, _CHUNK), jnp.int32),
            pltpu.VMEM((_NBUF, _CHUNK, 128), jnp.float32),
            pltpu.VMEM((dim, _CHUNK), jnp.float32),
            pltpu.VMEM((16, 128), jnp.float32),
            pltpu.SemaphoreType.DMA,
            pltpu.SemaphoreType.DMA,
        ],
    )
    def _gather(idx_hbm, s_hbm, tail_hbm, out_hbm,
                idx_all, t_v, rows_v, tr_v, tail_v, g0, g1):
        wid = lax.axis_index("s") * _NC + lax.axis_index("c")
        gs = (g0, g1)
        iota16 = lax.iota(jnp.int32, 16)

        pltpu.sync_copy(idx_hbm.at[wid], idx_all.at[pl.ds(0, k_max)])
        pltpu.sync_copy(tail_hbm, tail_v.at[pl.ds(0, trows)])

        def start_chunk(k, slot):
            for kk in range(_CHUNK // 16):
                v = idx_all.at[k][pl.ds(kk * 16, 16)]
                t_v[slot, pl.ds(kk * 16, 16)] = lax.min(
                    lax.shift_right_logical(v, jnp.int32(2)), jnp.int32(smax)
                )
            pltpu.async_copy(s_hbm.at[t_v.at[slot]], rows_v.at[slot], gs[slot])

        def finish_chunk(c, k, slot):
            pltpu.make_async_copy(
                s_hbm.at[t_v.at[slot]], rows_v.at[slot], gs[slot]
            ).wait()
            for kk in range(_CHUNK // 16):
                v = idx_all.at[k][pl.ds(kk * 16, 16)]
                coff = lax.shift_left(
                    lax.bitwise_and(v, jnp.int32(rpw - 1)), jnp.int32(5)
                )
                w = v - jnp.int32(vocab_cut)
                is_tail = v >= jnp.int32(vocab_cut)
                wr = lax.shift_right_logical(
                    lax.max(w, jnp.int32(0)), jnp.int32(2)
                )
                wcoff = lax.shift_left(
                    lax.bitwise_and(w, jnp.int32(rpw - 1)), jnp.int32(5)
                )
                ridx = kk * 16 + iota16
                for j in range(dim):
                    m = plsc.load_gather(rows_v.at[slot], [ridx, coff + j])
                    t = plsc.load_gather(tail_v, [wr, wcoff + j])
                    tr_v[j, pl.ds(kk * 16, 16)] = jnp.where(is_tail, t, m)
            pltpu.sync_copy(tr_v, out_hbm.at[:, pl.ds(c * _CHUNK, _CHUNK)])

        @pl.when(wid < n_chunks)
        def _():
            start_chunk(0, 0)

        @pl.loop(0, k_max, step=_NBUF)
        def _chunks(k):
            for b_ in range(_NBUF):
                c = wid + (k + b_) * _NW
                nxt = c + _NW
                slot = b_

                @pl.when(nxt < n_chunks)
                def _(k=k, b_=b_, slot=slot):
                    start_chunk(k + b_ + 1, (slot + 1) % _NBUF)

                @pl.when(c < n_chunks)
                def _(c=c, k=k, b_=b_, slot=slot):
                    finish_chunk(c, k + b_, slot)

    return _gather


def kernel(token_id, table):
    b = token_id.shape[0]
    vocab, dim = table.shape
    b_pad = -(-b // (_NW * _CHUNK)) * _NW * _CHUNK
    k_max = b_pad // (_NW * _CHUNK)
    idx = token_id.astype(jnp.int32)
    npad = b_pad - b
    if npad:
        pad = jnp.arange(npad, dtype=jnp.int32) % jnp.int32(vocab)
        idx = jnp.concatenate([idx, pad])
    idx3 = idx.reshape(k_max, _NW, _CHUNK).transpose(1, 0, 2)
    tt = table.T
    vocab_cut = (vocab // _TBLK) * _TBLK
    tail = table[vocab_cut:, :].reshape((vocab - vocab_cut) * dim // 128, 128)
    scratch = _build_transpose(vocab, dim)(tt)
    out_t = _build_gather(b_pad, vocab, dim)(idx3, scratch, tail)
    return out_t[:, :b].T


# speedup vs baseline: 1.4607x; 1.4607x over previous
import functools

import jax
import jax.numpy as jnp
from jax import lax
from jax.experimental import pallas as pl
from jax.experimental.pallas import tpu as pltpu
from jax.experimental.pallas import tpu_sc as plsc

_NC = 2
_NS = 16
_NW = _NC * _NS
_CHUNK = 128


@functools.lru_cache(maxsize=None)
def _build(n_chunks, vocab, dim):
    mesh = plsc.VectorSubcoreMesh(core_axis_name="c", subcore_axis_name="s")

    @functools.partial(
        pl.kernel,
        mesh=mesh,
        compiler_params=pltpu.CompilerParams(use_tc_tiling_on_sc=False),
        out_type=jax.ShapeDtypeStruct((_NW, n_chunks, _CHUNK, dim), jnp.float32),
        scratch_types=[
            pltpu.VMEM((n_chunks, _CHUNK), jnp.int32),
            pltpu.VMEM((n_chunks, _CHUNK, dim), jnp.float32),
            pltpu.SemaphoreType.DMA,
        ],
    )
    def _gather(idx_hbm, table_hbm, out_hbm, idx_v, rows_v, sem):
        wid = lax.axis_index("s") * _NC + lax.axis_index("c")
        pltpu.sync_copy(idx_hbm.at[wid], idx_v)
        copies = [
            pltpu.async_copy(table_hbm.at[idx_v.at[j]], rows_v.at[j], sem)
            for j in range(n_chunks)
        ]
        for c in copies:
            c.wait()
        pltpu.sync_copy(rows_v, out_hbm.at[wid])

    return _gather


def kernel(token_id, table):
    b = token_id.shape[0]
    vocab, dim = table.shape
    per_w = -(-b // (_NW * _CHUNK))
    b_pad = _NW * per_w * _CHUNK
    idx = token_id.astype(jnp.int32)
    npad = b_pad - b
    if npad:
        pad = jnp.arange(npad, dtype=jnp.int32) % jnp.int32(vocab)
        idx = jnp.concatenate([idx, pad])
    idx3 = idx.reshape(_NW, per_w, _CHUNK)
    out = _build(per_w, vocab, dim)(idx3, table)
    return out.reshape(b_pad, dim)[:b]
